# baseline (speedup 1.0000x reference)
import jax
import jax.numpy as jnp
from jax.experimental import pallas as pl
from jax.experimental.pallas import tpu as pltpu

_DIM = 32
_EPS = 0.01


def _fused_kernel(z_ref, x_ref, w1_ref, b1_ref, w2_ref, b2_ref,
                  prob_ref, lam_ref, h_ref, *, nf):
    j = pl.program_id(1)
    part = jnp.dot(x_ref[...], w1_ref[...],
                   preferred_element_type=jnp.float32)

    @pl.when(j == 0)
    def _():
        h_ref[...] = part

    @pl.when(j > 0)
    def _():
        h_ref[...] = h_ref[...] + part

    @pl.when(j == nf - 1)
    def _():
        zb = z_ref[...]
        h = jnp.maximum(h_ref[...] + b1_ref[...], 0.0)
        o = jnp.sum(h * w2_ref[...], axis=-1, keepdims=True) + b2_ref[0]
        ot = o.T
        lamt = jax.nn.log_sigmoid(ot) + (1.0 - _EPS)

        zt = zb.T
        k0 = jnp.zeros_like(zt)
        k1 = jnp.zeros_like(zt)
        s0 = jnp.zeros_like(zt)
        s1 = jnp.zeros_like(zt)
        for jj in range(0, _DIM, 2):
            zj0 = zt[jj:jj + 1, :]
            zj1 = zt[jj + 1:jj + 2, :]
            g0 = jnp.where(zj0 >= zt, 1.0, 0.0)
            g1 = jnp.where(zj1 >= zt, 1.0, 0.0)
            k0 = k0 + g0
            k1 = k1 + g1
            s0 = s0 + g0 * zj0
            s1 = s1 + g1 * zj1
        t_i = (s0 + s1 - 1.0 + lamt) / (k0 + k1)
        taut = jnp.max(t_i, axis=0, keepdims=True)
        denomt = jnp.maximum(1.0 - lamt, _EPS)
        probt = jnp.maximum(zt - taut, 0.0) / denomt
        prob_ref[...] = probt.T
        lam_ref[...] = lamt.T


def kernel(z, x, W1, b1, W2, b2):
    bs, seqlen, dim = z.shape
    n = bs * seqlen
    feat = x.shape[-1]
    hidden = W1.shape[0]
    zf = z.reshape(n, dim).astype(jnp.float32)
    xf = x.reshape(n, feat).astype(jnp.float32)
    w1t = W1.T
    b1r = b1.reshape(1, hidden)
    w2r = W2.reshape(1, hidden)

    tb = 2048
    fc = 256
    nf = feat // fc
    grid = (n // tb, nf)
    import functools
    prob, lam = pl.pallas_call(
        functools.partial(_fused_kernel, nf=nf),
        grid=grid,
        in_specs=[
            pl.BlockSpec((tb, dim), lambda i, j: (i, 0)),
            pl.BlockSpec((tb, fc), lambda i, j: (i, j)),
            pl.BlockSpec((fc, hidden), lambda i, j: (j, 0)),
            pl.BlockSpec((1, hidden), lambda i, j: (0, 0)),
            pl.BlockSpec((1, hidden), lambda i, j: (0, 0)),
            pl.BlockSpec(memory_space=pltpu.SMEM),
        ],
        out_specs=[
            pl.BlockSpec((tb, dim), lambda i, j: (i, 0)),
            pl.BlockSpec((tb, 1), lambda i, j: (i, 0)),
        ],
        out_shape=[
            jax.ShapeDtypeStruct((n, dim), jnp.float32),
            jax.ShapeDtypeStruct((n, 1), jnp.float32),
        ],
        scratch_shapes=[pltpu.VMEM((tb, hidden), jnp.float32)],
        compiler_params=pltpu.CompilerParams(
            dimension_semantics=("arbitrary", "arbitrary"),
        ),
    )(zf, xf, w1t, b1r, w2r, b2)
    return prob.reshape(bs, seqlen, dim), lam.reshape(bs, seqlen)


# speedup vs baseline: 1.6008x; 1.6008x over previous
import jax
import jax.numpy as jnp
from jax.experimental import pallas as pl
from jax.experimental.pallas import tpu as pltpu

_DIM = 32
_EPS = 0.01


def _fused_kernel(z_ref, x_ref, w1_ref, b1_ref, w2_ref, b2_ref,
                  prob_ref, lam_ref):
    xb = x_ref[...]
    zb = z_ref[...]
    tb = zb.shape[0]

    rw = 256
    o_parts = []
    for r in range(0, tb, rw):
        hr = jnp.dot(xb[r:r + rw, :], w1_ref[...],
                     preferred_element_type=jnp.float32)
        hr = jnp.maximum(hr + b1_ref[...], 0.0)
        o_parts.append(jnp.sum(hr * w2_ref[...], axis=-1, keepdims=True))
    o = jnp.concatenate(o_parts, axis=0) + b2_ref[0]
    ot = o.T
    lamt = jax.nn.log_sigmoid(ot) + (1.0 - _EPS)

    zt = zb.T
    cw = 512
    taut_parts = []
    for c in range(0, tb, cw):
        ztc = zt[:, c:c + cw]
        k0 = jnp.zeros_like(ztc)
        k1 = jnp.zeros_like(ztc)
        s0 = jnp.zeros_like(ztc)
        s1 = jnp.zeros_like(ztc)
        for j in range(0, _DIM, 2):
            zj0 = ztc[j:j + 1, :]
            zj1 = ztc[j + 1:j + 2, :]
            g0 = jnp.where(zj0 >= ztc, 1.0, 0.0)
            g1 = jnp.where(zj1 >= ztc, 1.0, 0.0)
            k0 = k0 + g0
            k1 = k1 + g1
            s0 = s0 + g0 * zj0
            s1 = s1 + g1 * zj1
        lamc = lamt[:, c:c + cw]
        t_i = (s0 + s1 - 1.0 + lamc) / (k0 + k1)
        taut_parts.append(jnp.max(t_i, axis=0, keepdims=True))
    taut = jnp.concatenate(taut_parts, axis=1)
    denomt = jnp.maximum(1.0 - lamt, _EPS)
    probt = jnp.maximum(zt - taut, 0.0) / denomt
    prob_ref[...] = probt.T
    lam_ref[...] = lamt.T


def kernel(z, x, W1, b1, W2, b2):
    bs, seqlen, dim = z.shape
    n = bs * seqlen
    feat = x.shape[-1]
    hidden = W1.shape[0]
    zf = z.reshape(n, dim).astype(jnp.float32)
    xf = x.reshape(n, feat).astype(jnp.float32)
    w1t = W1.T
    b1r = b1.reshape(1, hidden)
    w2r = W2.reshape(1, hidden)

    tb = 2048
    grid = (n // tb,)
    prob, lam = pl.pallas_call(
        _fused_kernel,
        grid=grid,
        in_specs=[
            pl.BlockSpec((tb, dim), lambda i: (i, 0)),
            pl.BlockSpec((tb, feat), lambda i: (i, 0)),
            pl.BlockSpec((feat, hidden), lambda i: (0, 0)),
            pl.BlockSpec((1, hidden), lambda i: (0, 0)),
            pl.BlockSpec((1, hidden), lambda i: (0, 0)),
            pl.BlockSpec(memory_space=pltpu.SMEM),
        ],
        out_specs=[
            pl.BlockSpec((tb, dim), lambda i: (i, 0)),
            pl.BlockSpec((tb, 1), lambda i: (i, 0)),
        ],
        out_shape=[
            jax.ShapeDtypeStruct((n, dim), jnp.float32),
            jax.ShapeDtypeStruct((n, 1), jnp.float32),
        ],
    )(zf, xf, w1t, b1r, w2r, b2)
    return prob.reshape(bs, seqlen, dim), lam.reshape(bs, seqlen)
